# baseline (speedup 1.0000x reference)
import functools

import jax
import jax.numpy as jnp
from jax import lax
from jax.experimental import pallas as pl
from jax.experimental.pallas import tpu as pltpu
from jax.experimental.pallas import tpu_sc as plsc

_N = 10000
_E = 320000
_NC = 2
_NS = 16
_NW = _NC * _NS
_EPW = _E // _NW
_K = 80
_NCHUNK = _EPW // _K
_NPAD = 10240
_RPT = _NPAD // _NS
_DUMMY = _N
_NBUF = 4
_BM = 1000


def _sc_mesh():
    return plsc.VectorSubcoreMesh(core_axis_name="c", subcore_axis_name="s")


def _sc_prep(row, col, zeros, ones):
    @functools.partial(
        pl.kernel,
        mesh=_sc_mesh(),
        out_type=[
            jax.ShapeDtypeStruct((_E,), jnp.int32),
            jax.ShapeDtypeStruct((_NC, _NPAD, 128), jnp.float32),
        ],
        scratch_types=(
            [pltpu.VMEM((_K,), jnp.int32)] * (3 * _NBUF)
            + [pltpu.VMEM((_K, 128), jnp.float32),
               pltpu.VMEM_SHARED((_NPAD, 128), jnp.float32)]
            + [pltpu.SemaphoreType.DMA] * (4 * _NBUF)
        ),
    )
    def k(row_hbm, col_hbm, z_hbm, ones_hbm, colp_hbm, degp_hbm, *scr):
        idxb = scr[:3 * _NBUF]
        ob = scr[3 * _NBUF]
        table = scr[3 * _NBUF + 1]
        sems = scr[3 * _NBUF + 2:]
        bufs = tuple(
            (idxb[3 * b], idxb[3 * b + 1], idxb[3 * b + 2],
             sems[4 * b], sems[4 * b + 1], sems[4 * b + 2], sems[4 * b + 3])
            for b in range(_NBUF))
        c = lax.axis_index("c")
        s = lax.axis_index("s")
        base = (c * _NS + s) * _EPW

        def start_idx(i, b):
            rb, cb, _, sr, sc, _, _ = bufs[b]
            off = base + i * _K
            pltpu.async_copy(row_hbm.at[pl.ds(off, _K)], rb, sr)
            pltpu.async_copy(col_hbm.at[pl.ds(off, _K)], cb, sc)

        def do_chunk(i, b):
            rb, cb, cpb, sr, sc, so, ss = bufs[b]
            off = base + i * _K
            pltpu.make_async_copy(row_hbm.at[pl.ds(0, _K)], rb, sr).wait()
            pltpu.make_async_copy(col_hbm.at[pl.ds(0, _K)], cb, sc).wait()
            for j in range(_K // 16):
                sl = pl.ds(j * 16, 16)
                r = rb[sl]
                cc = cb[sl]
                cpb[sl] = jnp.where(r == cc, _DUMMY, cc)
            pltpu.async_copy(cpb, colp_hbm.at[pl.ds(off, _K)], so)
            pltpu.async_copy(ob, table.at[cpb], ss, add=True)

        def drain(b):
            _, _, cpb, _, _, so, ss = bufs[b]
            pltpu.make_async_copy(cpb, colp_hbm.at[pl.ds(0, _K)], so).wait()
            pltpu.make_async_copy(ones_hbm, ob, ss).wait()

        pltpu.sync_copy(z_hbm.at[pl.ds(s * _RPT, _RPT)],
                        table.at[pl.ds(s * _RPT, _RPT)])
        pltpu.sync_copy(ones_hbm, ob)
        plsc.subcore_barrier()

        for b in range(_NBUF):
            start_idx(b, b)
        nfull = (_NCHUNK - 1) // _NBUF

        def body(p, carry):
            for b in range(_NBUF):
                do_chunk(p * _NBUF + b, b)
            for b in range(_NBUF):
                drain(b)
                nxt = p * _NBUF + b + _NBUF
                if b == 0:
                    start_idx(nxt, b)
                else:
                    @pl.when(nxt < _NCHUNK)
                    def _():
                        start_idx(nxt, b)
            return carry

        lax.fori_loop(0, nfull, body, 0)
        for i in range(nfull * _NBUF, _NCHUNK):
            do_chunk(i, i % _NBUF)
        for i in range(nfull * _NBUF, _NCHUNK):
            drain(i % _NBUF)
        plsc.subcore_barrier()
        pltpu.sync_copy(table.at[pl.ds(s * _RPT, _RPT)],
                        degp_hbm.at[c, pl.ds(s * _RPT, _RPT)])

    return k(row, col, zeros, ones)


def _sc_scatter(y, row, colp, zeros, d):
    @functools.partial(
        pl.kernel,
        mesh=_sc_mesh(),
        out_type=jax.ShapeDtypeStruct((_NC, _NPAD, d), jnp.float32),
        scratch_types=(
            [pltpu.VMEM((_K,), jnp.int32)] * (2 * _NBUF)
            + [pltpu.VMEM((_K, d), jnp.float32)] * _NBUF
            + [pltpu.VMEM_SHARED((_NPAD, d), jnp.float32)]
            + [pltpu.SemaphoreType.DMA] * (4 * _NBUF)
        ),
    )
    def k(y_hbm, row_hbm, colp_hbm, z_hbm, out_hbm, *scr):
        idxb = scr[:2 * _NBUF]
        gbs = scr[2 * _NBUF:3 * _NBUF]
        table = scr[3 * _NBUF]
        sems = scr[3 * _NBUF + 1:]
        bufs = tuple(
            (idxb[2 * b], idxb[2 * b + 1], gbs[b],
             sems[4 * b], sems[4 * b + 1], sems[4 * b + 2], sems[4 * b + 3])
            for b in range(_NBUF))
        c = lax.axis_index("c")
        s = lax.axis_index("s")
        base = (c * _NS + s) * _EPW

        def start_idx(i, b):
            rb, cpb, _, sr, sc, _, _ = bufs[b]
            off = base + i * _K
            pltpu.async_copy(row_hbm.at[pl.ds(off, _K)], rb, sr)
            pltpu.async_copy(colp_hbm.at[pl.ds(off, _K)], cpb, sc)

        def start_gather(b):
            rb, _, gb, sr, _, sg, _ = bufs[b]
            pltpu.make_async_copy(row_hbm.at[pl.ds(0, _K)], rb, sr).wait()
            pltpu.async_copy(y_hbm.at[rb], gb, sg)

        def start_scatter(b):
            _, cpb, gb, _, sc, sg, ss = bufs[b]
            pltpu.make_async_copy(y_hbm.at[pl.ds(0, _K)], gb, sg).wait()
            pltpu.make_async_copy(colp_hbm.at[pl.ds(0, _K)], cpb, sc).wait()
            pltpu.async_copy(gb, table.at[cpb], ss, add=True)

        def drain_scatter(b):
            _, _, gb, _, _, _, ss = bufs[b]
            pltpu.make_async_copy(y_hbm.at[pl.ds(0, _K)], gb, ss).wait()

        pltpu.sync_copy(z_hbm.at[pl.ds(s * _RPT, _RPT)],
                        table.at[pl.ds(s * _RPT, _RPT)])
        plsc.subcore_barrier()

        for b in range(_NBUF):
            start_idx(b, b)
        for b in range(_NBUF):
            start_gather(b)
        nfull = (_NCHUNK - 1) // _NBUF

        def body(p, carry):
            for b in range(_NBUF):
                start_scatter(b)
            for b in range(_NBUF):
                drain_scatter(b)
                nxt = p * _NBUF + b + _NBUF
                if b == 0:
                    start_idx(nxt, b)
                    start_gather(b)
                else:
                    @pl.when(nxt < _NCHUNK)
                    def _():
                        start_idx(nxt, b)
                        start_gather(b)
            return carry

        lax.fori_loop(0, nfull, body, 0)
        for i in range(nfull * _NBUF, _NCHUNK):
            start_scatter(i % _NBUF)
        for i in range(nfull * _NBUF, _NCHUNK):
            drain_scatter(i % _NBUF)
        plsc.subcore_barrier()
        pltpu.sync_copy(table.at[pl.ds(s * _RPT, _RPT)],
                        out_hbm.at[c, pl.ds(s * _RPT, _RPT)])

    return k(y, row, colp, zeros)


def _tc_stage1(degp, x, w, wp, a0, a1, ab, b0, b1, bb):
    def body(degp_ref, x_ref, w_ref, wp_r, a0_r, a1_r, ab_r, b0_r, b1_r,
             bb_r, y_ref, dis_ref, nrm_ref):
        d0 = degp_ref[0]
        d1 = degp_ref[1]
        deg = d0[:, 0:1] + d1[:, 0:1] + 1.0
        dis = lax.rsqrt(deg)
        y_ref[...] = dis * jnp.dot(x_ref[...], w_ref[...],
                                   preferred_element_type=jnp.float32)
        dis_ref[...] = dis

        @pl.when(pl.program_id(0) == 0)
        def _():
            h1 = jnp.dot(wp_r[...], a0_r[...],
                         preferred_element_type=jnp.float32)
            h1 = jnp.where(h1 >= 0, h1, 0.2 * h1)
            w1 = jnp.dot(h1, a1_r[...],
                         preferred_element_type=jnp.float32) + ab_r[...]
            w1 = jnp.where(w1 >= 0, w1, 0.01 * w1)
            h2 = jnp.dot(wp_r[...], b0_r[...],
                         preferred_element_type=jnp.float32)
            h2 = jnp.where(h2 >= 0, h2, 0.2 * h2)
            w2 = jnp.dot(h2, b1_r[...],
                         preferred_element_type=jnp.float32) + bb_r[...]
            w2 = jnp.where(w2 >= 0, w2, 0.01 * w2)
            nsq = (jnp.sum(w1 * w1, axis=1, keepdims=True)
                   + jnp.sum(w2 * w2, axis=1, keepdims=True))
            nrm_ref[...] = jnp.sqrt(nsq)

    full = lambda m: (0, 0)
    return pl.pallas_call(
        body,
        grid=(_N // _BM,),
        in_specs=[
            pl.BlockSpec((_NC, _BM, 128), lambda m: (0, m, 0)),
            pl.BlockSpec((_BM, 128), lambda m: (m, 0)),
            pl.BlockSpec((128, 128), full),
            pl.BlockSpec((4, 8), full),
            pl.BlockSpec((8, 128), full),
            pl.BlockSpec((128, 128), full),
            pl.BlockSpec((1, 128), full),
            pl.BlockSpec((8, 64), full),
            pl.BlockSpec((64, 64), full),
            pl.BlockSpec((1, 64), full),
        ],
        out_specs=[
            pl.BlockSpec((_BM, 128), lambda m: (m, 0)),
            pl.BlockSpec((_BM, 1), lambda m: (m, 0)),
            pl.BlockSpec((4, 1), full),
        ],
        out_shape=[
            jax.ShapeDtypeStruct((_N, 128), jnp.float32),
            jax.ShapeDtypeStruct((_N, 1), jnp.float32),
            jax.ShapeDtypeStruct((4, 1), jnp.float32),
        ],
    )(degp, x, w, wp, a0, a1, ab, b0, b1, bb)


def _tc_stage2(sp, y1, dis, b1, w2):
    def body(sp_ref, y1_ref, dis_ref, b1_ref, w2_ref, y2_ref):
        o = dis_ref[...] * (sp_ref[0] + sp_ref[1] + y1_ref[...]) + b1_ref[...]
        h = jnp.where(o > 0, o, jnp.exp(jnp.minimum(o, 0.0)) - 1.0)
        y2 = dis_ref[...] * jnp.dot(h, w2_ref[...],
                                    preferred_element_type=jnp.float32)
        y2_ref[...] = jnp.concatenate(
            [y2, jnp.zeros((_BM, 64), jnp.float32)], axis=1)

    return pl.pallas_call(
        body,
        grid=(_N // _BM,),
        in_specs=[
            pl.BlockSpec((_NC, _BM, 128), lambda m: (0, m, 0)),
            pl.BlockSpec((_BM, 128), lambda m: (m, 0)),
            pl.BlockSpec((_BM, 1), lambda m: (m, 0)),
            pl.BlockSpec((1, 128), lambda m: (0, 0)),
            pl.BlockSpec((128, 64), lambda m: (0, 0)),
        ],
        out_specs=pl.BlockSpec((_BM, 128), lambda m: (m, 0)),
        out_shape=jax.ShapeDtypeStruct((_N, 128), jnp.float32),
    )(sp, y1, dis, b1, w2)


def _tc_stage3(sp, y2, dis, b2):
    def body(sp_ref, y2_ref, dis_ref, b2_ref, out_ref):
        t = sp_ref[0] + sp_ref[1] + y2_ref[...]
        o = dis_ref[...] * t[:, :64] + b2_ref[...]
        m = jnp.max(o, axis=1, keepdims=True)
        e = jnp.exp(o - m)
        out_ref[...] = (o - m) - jnp.log(jnp.sum(e, axis=1, keepdims=True))

    return pl.pallas_call(
        body,
        grid=(_N // _BM,),
        in_specs=[
            pl.BlockSpec((_NC, _BM, 128), lambda m: (0, m, 0)),
            pl.BlockSpec((_BM, 128), lambda m: (m, 0)),
            pl.BlockSpec((_BM, 1), lambda m: (m, 0)),
            pl.BlockSpec((1, 64), lambda m: (0, 0)),
        ],
        out_specs=pl.BlockSpec((_BM, 64), lambda m: (m, 0)),
        out_shape=jax.ShapeDtypeStruct((_N, 64), jnp.float32),
    )(sp, y2, dis, b2)


def kernel(x, edge_index, w_mul_p, lin1_w, bias1, mlp1_w0, mlp1_w1, mlp1_b1,
           lin2_w, bias2, mlp2_w0, mlp2_w1, mlp2_b1):
    zeros128 = jnp.zeros((_NPAD, 128), jnp.float32)
    ones_k = jnp.ones((_K, 128), jnp.float32)

    row = edge_index[0]
    col = edge_index[1]
    colp, degp = _sc_prep(row, col, zeros128, ones_k)
    y1, dis, nrm = _tc_stage1(degp, x, lin1_w, w_mul_p, mlp1_w0, mlp1_w1,
                              mlp1_b1.reshape(1, 128), mlp2_w0, mlp2_w1,
                              mlp2_b1.reshape(1, 64))
    s1 = _sc_scatter(y1, row, colp, zeros128, 128)
    y2 = _tc_stage2(s1, y1, dis, bias1.reshape(1, 128), lin2_w)
    s2 = _sc_scatter(y2, row, colp, zeros128, 128)
    logp = _tc_stage3(s2, y2, dis, bias2.reshape(1, 64))
    return logp, nrm.reshape(4)


# speedup vs baseline: 1.0008x; 1.0008x over previous
import functools

import jax
import jax.numpy as jnp
from jax import lax
from jax.experimental import pallas as pl
from jax.experimental.pallas import tpu as pltpu
from jax.experimental.pallas import tpu_sc as plsc

_N = 10000
_E = 320000
_NC = 2
_NS = 16
_NW = _NC * _NS
_EPW = _E // _NW
_K = 80
_NCHUNK = _EPW // _K
_NPAD = 10240
_RPT = _NPAD // _NS
_DUMMY = _N
_NBUF = 4
_BM = 1000


def _sc_mesh():
    return plsc.VectorSubcoreMesh(core_axis_name="c", subcore_axis_name="s")


def _sc_prep(row, col, zeros, ones):
    @functools.partial(
        pl.kernel,
        mesh=_sc_mesh(),
        out_type=[
            jax.ShapeDtypeStruct((_E,), jnp.int32),
            jax.ShapeDtypeStruct((_NC, _NPAD, 128), jnp.float32),
        ],
        scratch_types=(
            [pltpu.VMEM((_K,), jnp.int32)] * (3 * _NBUF)
            + [pltpu.VMEM((_K, 128), jnp.float32),
               pltpu.VMEM_SHARED((_NPAD, 128), jnp.float32)]
            + [pltpu.SemaphoreType.DMA] * (4 * _NBUF)
        ),
    )
    def k(row_hbm, col_hbm, z_hbm, ones_hbm, colp_hbm, degp_hbm, *scr):
        idxb = scr[:3 * _NBUF]
        ob = scr[3 * _NBUF]
        table = scr[3 * _NBUF + 1]
        sems = scr[3 * _NBUF + 2:]
        bufs = tuple(
            (idxb[3 * b], idxb[3 * b + 1], idxb[3 * b + 2],
             sems[4 * b], sems[4 * b + 1], sems[4 * b + 2], sems[4 * b + 3])
            for b in range(_NBUF))
        c = lax.axis_index("c")
        s = lax.axis_index("s")
        base = (c * _NS + s) * _EPW

        def start_idx(i, b):
            rb, cb, _, sr, sc, _, _ = bufs[b]
            off = base + i * _K
            pltpu.async_copy(row_hbm.at[pl.ds(off, _K)], rb, sr)
            pltpu.async_copy(col_hbm.at[pl.ds(off, _K)], cb, sc)

        def do_chunk(i, b):
            rb, cb, cpb, sr, sc, so, ss = bufs[b]
            off = base + i * _K
            pltpu.make_async_copy(row_hbm.at[pl.ds(0, _K)], rb, sr).wait()
            pltpu.make_async_copy(col_hbm.at[pl.ds(0, _K)], cb, sc).wait()
            for j in range(_K // 16):
                sl = pl.ds(j * 16, 16)
                r = rb[sl]
                cc = cb[sl]
                cpb[sl] = jnp.where(r == cc, _DUMMY, cc)
            pltpu.async_copy(cpb, colp_hbm.at[pl.ds(off, _K)], so)
            pltpu.async_copy(ob, table.at[cpb], ss, add=True)

        def drain(b):
            _, _, cpb, _, _, so, ss = bufs[b]
            pltpu.make_async_copy(cpb, colp_hbm.at[pl.ds(0, _K)], so).wait()
            pltpu.make_async_copy(ones_hbm, ob, ss).wait()

        pltpu.sync_copy(z_hbm.at[pl.ds(s * _RPT, _RPT)],
                        table.at[pl.ds(s * _RPT, _RPT)])
        pltpu.sync_copy(ones_hbm, ob)
        plsc.subcore_barrier()

        for b in range(_NBUF):
            start_idx(b, b)
        nfull = (_NCHUNK - 1) // _NBUF

        def body(p, carry):
            for b in range(_NBUF):
                do_chunk(p * _NBUF + b, b)
            for b in range(_NBUF):
                drain(b)
                nxt = p * _NBUF + b + _NBUF
                if b == 0:
                    start_idx(nxt, b)
                else:
                    @pl.when(nxt < _NCHUNK)
                    def _():
                        start_idx(nxt, b)
            return carry

        lax.fori_loop(0, nfull, body, 0)
        for i in range(nfull * _NBUF, _NCHUNK):
            do_chunk(i, i % _NBUF)
        for i in range(nfull * _NBUF, _NCHUNK):
            drain(i % _NBUF)
        plsc.subcore_barrier()
        pltpu.sync_copy(table.at[pl.ds(s * _RPT, _RPT)],
                        degp_hbm.at[c, pl.ds(s * _RPT, _RPT)])

    return k(row, col, zeros, ones)


def _sc_scatter(y, row, colp, zeros, d):
    @functools.partial(
        pl.kernel,
        mesh=_sc_mesh(),
        out_type=jax.ShapeDtypeStruct((_NC, _NPAD, d), jnp.float32),
        scratch_types=(
            [pltpu.VMEM((_K,), jnp.int32)] * (2 * _NBUF)
            + [pltpu.VMEM((_K, d), jnp.float32)] * _NBUF
            + [pltpu.VMEM_SHARED((_NPAD, d), jnp.float32)]
            + [pltpu.SemaphoreType.DMA] * (4 * _NBUF)
        ),
    )
    def k(y_hbm, row_hbm, colp_hbm, z_hbm, out_hbm, *scr):
        idxb = scr[:2 * _NBUF]
        gbs = scr[2 * _NBUF:3 * _NBUF]
        table = scr[3 * _NBUF]
        sems = scr[3 * _NBUF + 1:]
        bufs = tuple(
            (idxb[2 * b], idxb[2 * b + 1], gbs[b],
             sems[4 * b], sems[4 * b + 1], sems[4 * b + 2], sems[4 * b + 3])
            for b in range(_NBUF))
        c = lax.axis_index("c")
        s = lax.axis_index("s")
        base = (c * _NS + s) * _EPW

        def start_idx(i, b):
            rb, cpb, _, sr, sc, _, _ = bufs[b]
            off = base + i * _K
            pltpu.async_copy(row_hbm.at[pl.ds(off, _K)], rb, sr)
            pltpu.async_copy(colp_hbm.at[pl.ds(off, _K)], cpb, sc)

        def start_gather(b):
            rb, _, gb, sr, _, sg, _ = bufs[b]
            pltpu.make_async_copy(row_hbm.at[pl.ds(0, _K)], rb, sr).wait()
            pltpu.async_copy(y_hbm.at[rb], gb, sg)

        def start_scatter(b):
            _, cpb, gb, _, sc, sg, ss = bufs[b]
            pltpu.make_async_copy(y_hbm.at[pl.ds(0, _K)], gb, sg).wait()
            pltpu.make_async_copy(colp_hbm.at[pl.ds(0, _K)], cpb, sc).wait()
            pltpu.async_copy(gb, table.at[cpb], ss, add=True)

        def drain_scatter(b):
            _, _, gb, _, _, _, ss = bufs[b]
            pltpu.make_async_copy(y_hbm.at[pl.ds(0, _K)], gb, ss).wait()

        pltpu.sync_copy(z_hbm.at[pl.ds(s * _RPT, _RPT)],
                        table.at[pl.ds(s * _RPT, _RPT)])
        plsc.subcore_barrier()

        for b in range(_NBUF):
            start_idx(b, b)
        for b in range(_NBUF):
            start_gather(b)
        nfull = (_NCHUNK - 1) // _NBUF

        def body(p, carry):
            for b in range(_NBUF):
                start_scatter(b)
            for b in range(_NBUF):
                drain_scatter(b)
                nxt = p * _NBUF + b + _NBUF
                if b == 0:
                    start_idx(nxt, b)
                    start_gather(b)
                else:
                    @pl.when(nxt < _NCHUNK)
                    def _():
                        start_idx(nxt, b)
                        start_gather(b)
            return carry

        lax.fori_loop(0, nfull, body, 0)
        for i in range(nfull * _NBUF, _NCHUNK):
            start_scatter(i % _NBUF)
        for i in range(nfull * _NBUF, _NCHUNK):
            drain_scatter(i % _NBUF)
        plsc.subcore_barrier()
        pltpu.sync_copy(table.at[pl.ds(s * _RPT, _RPT)],
                        out_hbm.at[c, pl.ds(s * _RPT, _RPT)])

    return k(y, row, colp, zeros)


def _tc_stage1(degp, x, w):
    def body(degp_ref, x_ref, w_ref, y_ref, dis_ref):
        d0 = degp_ref[0]
        d1 = degp_ref[1]
        deg = d0[:, 0:1] + d1[:, 0:1] + 1.0
        dis = lax.rsqrt(deg)
        y_ref[...] = dis * jnp.dot(x_ref[...], w_ref[...],
                                   preferred_element_type=jnp.float32)
        dis_ref[...] = dis

    return pl.pallas_call(
        body,
        grid=(_N // _BM,),
        in_specs=[
            pl.BlockSpec((_NC, _BM, 128), lambda m: (0, m, 0)),
            pl.BlockSpec((_BM, 128), lambda m: (m, 0)),
            pl.BlockSpec((128, 128), lambda m: (0, 0)),
        ],
        out_specs=[
            pl.BlockSpec((_BM, 128), lambda m: (m, 0)),
            pl.BlockSpec((_BM, 1), lambda m: (m, 0)),
        ],
        out_shape=[
            jax.ShapeDtypeStruct((_N, 128), jnp.float32),
            jax.ShapeDtypeStruct((_N, 1), jnp.float32),
        ],
    )(degp, x, w)


def _tc_pweight(wp, a0, a1, ab, b0, b1, bb):
    def body(wp_r, a0_r, a1_r, ab_r, b0_r, b1_r, bb_r, out_ref):
        h1 = jnp.dot(wp_r[...], a0_r[...], preferred_element_type=jnp.float32)
        h1 = jnp.where(h1 >= 0, h1, 0.2 * h1)
        w1 = jnp.dot(h1, a1_r[...], preferred_element_type=jnp.float32) + ab_r[...]
        w1 = jnp.where(w1 >= 0, w1, 0.01 * w1)
        h2 = jnp.dot(wp_r[...], b0_r[...], preferred_element_type=jnp.float32)
        h2 = jnp.where(h2 >= 0, h2, 0.2 * h2)
        w2 = jnp.dot(h2, b1_r[...], preferred_element_type=jnp.float32) + bb_r[...]
        w2 = jnp.where(w2 >= 0, w2, 0.01 * w2)
        nsq = (jnp.sum(w1 * w1, axis=1, keepdims=True)
               + jnp.sum(w2 * w2, axis=1, keepdims=True))
        out_ref[...] = jnp.sqrt(nsq)

    return pl.pallas_call(
        body,
        out_shape=jax.ShapeDtypeStruct((4, 1), jnp.float32),
    )(wp, a0, a1, ab, b0, b1, bb)


def _tc_stage2(sp, y1, dis, b1, w2):
    def body(sp_ref, y1_ref, dis_ref, b1_ref, w2_ref, y2_ref):
        o = dis_ref[...] * (sp_ref[0] + sp_ref[1] + y1_ref[...]) + b1_ref[...]
        h = jnp.where(o > 0, o, jnp.exp(jnp.minimum(o, 0.0)) - 1.0)
        y2 = dis_ref[...] * jnp.dot(h, w2_ref[...],
                                    preferred_element_type=jnp.float32)
        y2_ref[...] = jnp.concatenate(
            [y2, jnp.zeros((_BM, 64), jnp.float32)], axis=1)

    return pl.pallas_call(
        body,
        grid=(_N // _BM,),
        in_specs=[
            pl.BlockSpec((_NC, _BM, 128), lambda m: (0, m, 0)),
            pl.BlockSpec((_BM, 128), lambda m: (m, 0)),
            pl.BlockSpec((_BM, 1), lambda m: (m, 0)),
            pl.BlockSpec((1, 128), lambda m: (0, 0)),
            pl.BlockSpec((128, 64), lambda m: (0, 0)),
        ],
        out_specs=pl.BlockSpec((_BM, 128), lambda m: (m, 0)),
        out_shape=jax.ShapeDtypeStruct((_N, 128), jnp.float32),
    )(sp, y1, dis, b1, w2)


def _tc_stage3(sp, y2, dis, b2):
    def body(sp_ref, y2_ref, dis_ref, b2_ref, out_ref):
        t = sp_ref[0] + sp_ref[1] + y2_ref[...]
        o = dis_ref[...] * t[:, :64] + b2_ref[...]
        m = jnp.max(o, axis=1, keepdims=True)
        e = jnp.exp(o - m)
        out_ref[...] = (o - m) - jnp.log(jnp.sum(e, axis=1, keepdims=True))

    return pl.pallas_call(
        body,
        grid=(_N // _BM,),
        in_specs=[
            pl.BlockSpec((_NC, _BM, 128), lambda m: (0, m, 0)),
            pl.BlockSpec((_BM, 128), lambda m: (m, 0)),
            pl.BlockSpec((_BM, 1), lambda m: (m, 0)),
            pl.BlockSpec((1, 64), lambda m: (0, 0)),
        ],
        out_specs=pl.BlockSpec((_BM, 64), lambda m: (m, 0)),
        out_shape=jax.ShapeDtypeStruct((_N, 64), jnp.float32),
    )(sp, y2, dis, b2)


def kernel(x, edge_index, w_mul_p, lin1_w, bias1, mlp1_w0, mlp1_w1, mlp1_b1,
           lin2_w, bias2, mlp2_w0, mlp2_w1, mlp2_b1):
    zeros128 = jnp.zeros((_NPAD, 128), jnp.float32)
    ones_k = jnp.ones((_K, 128), jnp.float32)

    row = edge_index[0]
    col = edge_index[1]
    colp, degp = _sc_prep(row, col, zeros128, ones_k)
    y1, dis = _tc_stage1(degp, x, lin1_w)
    s1 = _sc_scatter(y1, row, colp, zeros128, 128)
    y2 = _tc_stage2(s1, y1, dis, bias1.reshape(1, 128), lin2_w)
    s2 = _sc_scatter(y2, row, colp, zeros128, 128)
    logp = _tc_stage3(s2, y2, dis, bias2.reshape(1, 64))
    norms = _tc_pweight(w_mul_p, mlp1_w0, mlp1_w1, mlp1_b1.reshape(1, 128),
                        mlp2_w0, mlp2_w1, mlp2_b1.reshape(1, 64)).reshape(4)
    return logp, norms


# speedup vs baseline: 1.0219x; 1.0210x over previous
import functools

import jax
import jax.numpy as jnp
from jax import lax
from jax.experimental import pallas as pl
from jax.experimental.pallas import tpu as pltpu
from jax.experimental.pallas import tpu_sc as plsc

_N = 10000
_E = 320000
_NC = 2
_NS = 16
_NW = _NC * _NS
_EPW = _E // _NW
_K = 80
_NCHUNK = _EPW // _K
_NPAD = 10240
_RPT = _NPAD // _NS
_DUMMY = _N
_NBUF = 4
_BM = 1000


def _sc_mesh():
    return plsc.VectorSubcoreMesh(core_axis_name="c", subcore_axis_name="s")


def _sc_prep(row, col, zeros, ones):
    @functools.partial(
        pl.kernel,
        mesh=_sc_mesh(),
        out_type=[
            jax.ShapeDtypeStruct((_E,), jnp.int32),
            jax.ShapeDtypeStruct((_NC, _NPAD, 128), jnp.float32),
        ],
        scratch_types=(
            [pltpu.VMEM((_K,), jnp.int32)] * (3 * _NBUF)
            + [pltpu.VMEM((_K, 128), jnp.float32),
               pltpu.VMEM_SHARED((_NPAD, 128), jnp.float32)]
            + [pltpu.SemaphoreType.DMA] * (4 * _NBUF)
        ),
    )
    def k(row_hbm, col_hbm, z_hbm, ones_hbm, colp_hbm, degp_hbm, *scr):
        idxb = scr[:3 * _NBUF]
        ob = scr[3 * _NBUF]
        table = scr[3 * _NBUF + 1]
        sems = scr[3 * _NBUF + 2:]
        bufs = tuple(
            (idxb[3 * b], idxb[3 * b + 1], idxb[3 * b + 2],
             sems[4 * b], sems[4 * b + 1], sems[4 * b + 2], sems[4 * b + 3])
            for b in range(_NBUF))
        c = lax.axis_index("c")
        s = lax.axis_index("s")
        base = (c * _NS + s) * _EPW

        def start_idx(i, b):
            rb, cb, _, sr, sc, _, _ = bufs[b]
            off = base + i * _K
            pltpu.async_copy(row_hbm.at[pl.ds(off, _K)], rb, sr)
            pltpu.async_copy(col_hbm.at[pl.ds(off, _K)], cb, sc)

        def do_chunk(i, b):
            rb, cb, cpb, sr, sc, so, ss = bufs[b]
            off = base + i * _K
            pltpu.make_async_copy(row_hbm.at[pl.ds(0, _K)], rb, sr).wait()
            pltpu.make_async_copy(col_hbm.at[pl.ds(0, _K)], cb, sc).wait()
            for j in range(_K // 16):
                sl = pl.ds(j * 16, 16)
                r = rb[sl]
                cc = cb[sl]
                cpb[sl] = jnp.where(r == cc, _DUMMY, cc)
            pltpu.async_copy(cpb, colp_hbm.at[pl.ds(off, _K)], so)
            pltpu.async_copy(ob, table.at[cpb], ss, add=True)

        def drain(b):
            _, _, cpb, _, _, so, ss = bufs[b]
            pltpu.make_async_copy(cpb, colp_hbm.at[pl.ds(0, _K)], so).wait()
            pltpu.make_async_copy(ones_hbm, ob, ss).wait()

        pltpu.sync_copy(z_hbm.at[pl.ds(s * _RPT, _RPT)],
                        table.at[pl.ds(s * _RPT, _RPT)])
        pltpu.sync_copy(ones_hbm, ob)
        plsc.subcore_barrier()

        for b in range(_NBUF):
            start_idx(b, b)
        nfull = (_NCHUNK - 1) // _NBUF

        def body(p, carry):
            for b in range(_NBUF):
                do_chunk(p * _NBUF + b, b)
            for b in range(_NBUF):
                drain(b)
                nxt = p * _NBUF + b + _NBUF
                if b == 0:
                    start_idx(nxt, b)
                else:
                    @pl.when(nxt < _NCHUNK)
                    def _():
                        start_idx(nxt, b)
            return carry

        lax.fori_loop(0, nfull, body, 0)
        for i in range(nfull * _NBUF, _NCHUNK):
            do_chunk(i, i % _NBUF)
        for i in range(nfull * _NBUF, _NCHUNK):
            drain(i % _NBUF)
        plsc.subcore_barrier()
        pltpu.sync_copy(table.at[pl.ds(s * _RPT, _RPT)],
                        degp_hbm.at[c, pl.ds(s * _RPT, _RPT)])

    return k(row, col, zeros, ones)


def _sc_scatter(y, row, colp, zeros, d, untiled=False):
    @functools.partial(
        pl.kernel,
        mesh=_sc_mesh(),
        compiler_params=(pltpu.CompilerParams(use_tc_tiling_on_sc=False)
                         if untiled else None),
        out_type=jax.ShapeDtypeStruct((_NC, _NPAD, d), jnp.float32),
        scratch_types=(
            [pltpu.VMEM((_K,), jnp.int32)] * (2 * _NBUF)
            + [pltpu.VMEM((_K, d), jnp.float32)] * _NBUF
            + [pltpu.VMEM_SHARED((_NPAD, d), jnp.float32)]
            + [pltpu.SemaphoreType.DMA] * (4 * _NBUF)
        ),
    )
    def k(y_hbm, row_hbm, colp_hbm, z_hbm, out_hbm, *scr):
        idxb = scr[:2 * _NBUF]
        gbs = scr[2 * _NBUF:3 * _NBUF]
        table = scr[3 * _NBUF]
        sems = scr[3 * _NBUF + 1:]
        bufs = tuple(
            (idxb[2 * b], idxb[2 * b + 1], gbs[b],
             sems[4 * b], sems[4 * b + 1], sems[4 * b + 2], sems[4 * b + 3])
            for b in range(_NBUF))
        c = lax.axis_index("c")
        s = lax.axis_index("s")
        base = (c * _NS + s) * _EPW

        def start_idx(i, b):
            rb, cpb, _, sr, sc, _, _ = bufs[b]
            off = base + i * _K
            pltpu.async_copy(row_hbm.at[pl.ds(off, _K)], rb, sr)
            pltpu.async_copy(colp_hbm.at[pl.ds(off, _K)], cpb, sc)

        def start_gather(b):
            rb, _, gb, sr, _, sg, _ = bufs[b]
            pltpu.make_async_copy(row_hbm.at[pl.ds(0, _K)], rb, sr).wait()
            pltpu.async_copy(y_hbm.at[rb], gb, sg)

        def start_scatter(b):
            _, cpb, gb, _, sc, sg, ss = bufs[b]
            pltpu.make_async_copy(y_hbm.at[pl.ds(0, _K)], gb, sg).wait()
            pltpu.make_async_copy(colp_hbm.at[pl.ds(0, _K)], cpb, sc).wait()
            pltpu.async_copy(gb, table.at[cpb], ss, add=True)

        def drain_scatter(b):
            _, _, gb, _, _, _, ss = bufs[b]
            pltpu.make_async_copy(y_hbm.at[pl.ds(0, _K)], gb, ss).wait()

        pltpu.sync_copy(z_hbm.at[pl.ds(s * _RPT, _RPT)],
                        table.at[pl.ds(s * _RPT, _RPT)])
        plsc.subcore_barrier()

        for b in range(_NBUF):
            start_idx(b, b)
        for b in range(_NBUF):
            start_gather(b)
        nfull = (_NCHUNK - 1) // _NBUF

        def body(p, carry):
            for b in range(_NBUF):
                start_scatter(b)
            for b in range(_NBUF):
                drain_scatter(b)
                nxt = p * _NBUF + b + _NBUF
                if b == 0:
                    start_idx(nxt, b)
                    start_gather(b)
                else:
                    @pl.when(nxt < _NCHUNK)
                    def _():
                        start_idx(nxt, b)
                        start_gather(b)
            return carry

        lax.fori_loop(0, nfull, body, 0)
        for i in range(nfull * _NBUF, _NCHUNK):
            start_scatter(i % _NBUF)
        for i in range(nfull * _NBUF, _NCHUNK):
            drain_scatter(i % _NBUF)
        plsc.subcore_barrier()
        pltpu.sync_copy(table.at[pl.ds(s * _RPT, _RPT)],
                        out_hbm.at[c, pl.ds(s * _RPT, _RPT)])

    return k(y, row, colp, zeros)


def _tc_stage1(degp, x, w):
    def body(degp_ref, x_ref, w_ref, y_ref, dis_ref):
        d0 = degp_ref[0]
        d1 = degp_ref[1]
        deg = d0[:, 0:1] + d1[:, 0:1] + 1.0
        dis = lax.rsqrt(deg)
        y_ref[...] = dis * jnp.dot(x_ref[...], w_ref[...],
                                   preferred_element_type=jnp.float32)
        dis_ref[...] = dis

    return pl.pallas_call(
        body,
        grid=(_N // _BM,),
        in_specs=[
            pl.BlockSpec((_NC, _BM, 128), lambda m: (0, m, 0)),
            pl.BlockSpec((_BM, 128), lambda m: (m, 0)),
            pl.BlockSpec((128, 128), lambda m: (0, 0)),
        ],
        out_specs=[
            pl.BlockSpec((_BM, 128), lambda m: (m, 0)),
            pl.BlockSpec((_BM, 1), lambda m: (m, 0)),
        ],
        out_shape=[
            jax.ShapeDtypeStruct((_N, 128), jnp.float32),
            jax.ShapeDtypeStruct((_N, 1), jnp.float32),
        ],
    )(degp, x, w)


def _tc_pweight(wp, a0, a1, ab, b0, b1, bb):
    def body(wp_r, a0_r, a1_r, ab_r, b0_r, b1_r, bb_r, out_ref):
        h1 = jnp.dot(wp_r[...], a0_r[...], preferred_element_type=jnp.float32)
        h1 = jnp.where(h1 >= 0, h1, 0.2 * h1)
        w1 = jnp.dot(h1, a1_r[...], preferred_element_type=jnp.float32) + ab_r[...]
        w1 = jnp.where(w1 >= 0, w1, 0.01 * w1)
        h2 = jnp.dot(wp_r[...], b0_r[...], preferred_element_type=jnp.float32)
        h2 = jnp.where(h2 >= 0, h2, 0.2 * h2)
        w2 = jnp.dot(h2, b1_r[...], preferred_element_type=jnp.float32) + bb_r[...]
        w2 = jnp.where(w2 >= 0, w2, 0.01 * w2)
        nsq = (jnp.sum(w1 * w1, axis=1, keepdims=True)
               + jnp.sum(w2 * w2, axis=1, keepdims=True))
        out_ref[...] = jnp.sqrt(nsq)

    return pl.pallas_call(
        body,
        out_shape=jax.ShapeDtypeStruct((4, 1), jnp.float32),
    )(wp, a0, a1, ab, b0, b1, bb)


def _tc_stage2(sp, y1, dis, b1, w2):
    def body(sp_ref, y1_ref, dis_ref, b1_ref, w2_ref, y2_ref):
        o = dis_ref[...] * (sp_ref[0] + sp_ref[1] + y1_ref[...]) + b1_ref[...]
        h = jnp.where(o > 0, o, jnp.exp(jnp.minimum(o, 0.0)) - 1.0)
        y2_ref[...] = dis_ref[...] * jnp.dot(h, w2_ref[...],
                                             preferred_element_type=jnp.float32)

    return pl.pallas_call(
        body,
        grid=(_N // _BM,),
        in_specs=[
            pl.BlockSpec((_NC, _BM, 128), lambda m: (0, m, 0)),
            pl.BlockSpec((_BM, 128), lambda m: (m, 0)),
            pl.BlockSpec((_BM, 1), lambda m: (m, 0)),
            pl.BlockSpec((1, 128), lambda m: (0, 0)),
            pl.BlockSpec((128, 64), lambda m: (0, 0)),
        ],
        out_specs=pl.BlockSpec((_BM, 64), lambda m: (m, 0)),
        out_shape=jax.ShapeDtypeStruct((_N, 64), jnp.float32),
    )(sp, y1, dis, b1, w2)


def _tc_stage3(sp, y2, dis, b2):
    def body(sp_ref, y2_ref, dis_ref, b2_ref, out_ref):
        t = sp_ref[0] + sp_ref[1] + y2_ref[...]
        o = dis_ref[...] * t + b2_ref[...]
        m = jnp.max(o, axis=1, keepdims=True)
        e = jnp.exp(o - m)
        out_ref[...] = (o - m) - jnp.log(jnp.sum(e, axis=1, keepdims=True))

    return pl.pallas_call(
        body,
        grid=(_N // _BM,),
        in_specs=[
            pl.BlockSpec((_NC, _BM, 64), lambda m: (0, m, 0)),
            pl.BlockSpec((_BM, 64), lambda m: (m, 0)),
            pl.BlockSpec((_BM, 1), lambda m: (m, 0)),
            pl.BlockSpec((1, 64), lambda m: (0, 0)),
        ],
        out_specs=pl.BlockSpec((_BM, 64), lambda m: (m, 0)),
        out_shape=jax.ShapeDtypeStruct((_N, 64), jnp.float32),
    )(sp, y2, dis, b2)


def kernel(x, edge_index, w_mul_p, lin1_w, bias1, mlp1_w0, mlp1_w1, mlp1_b1,
           lin2_w, bias2, mlp2_w0, mlp2_w1, mlp2_b1):
    zeros128 = jnp.zeros((_NPAD, 128), jnp.float32)
    zeros64 = jnp.zeros((_NPAD, 64), jnp.float32)
    ones_k = jnp.ones((_K, 128), jnp.float32)

    row = edge_index[0]
    col = edge_index[1]
    colp, degp = _sc_prep(row, col, zeros128, ones_k)
    y1, dis = _tc_stage1(degp, x, lin1_w)
    s1 = _sc_scatter(y1, row, colp, zeros128, 128)
    y2 = _tc_stage2(s1, y1, dis, bias1.reshape(1, 128), lin2_w)
    s2 = _sc_scatter(y2, row, colp, zeros64, 64, untiled=True)
    logp = _tc_stage3(s2, y2, dis, bias2.reshape(1, 64))
    norms = _tc_pweight(w_mul_p, mlp1_w0, mlp1_w1, mlp1_b1.reshape(1, 128),
                        mlp2_w0, mlp2_w1, mlp2_b1.reshape(1, 64)).reshape(4)
    return logp, norms


# speedup vs baseline: 1.0296x; 1.0075x over previous
import functools

import jax
import jax.numpy as jnp
from jax import lax
from jax.experimental import pallas as pl
from jax.experimental.pallas import tpu as pltpu
from jax.experimental.pallas import tpu_sc as plsc

_N = 10000
_E = 320000
_NC = 2
_NS = 16
_NW = _NC * _NS
_EPW = _E // _NW
_K = 80
_NCHUNK = _EPW // _K
_NPAD = 10240
_RPT = _NPAD // _NS
_DUMMY = _N
_NBUF = 4
_BM = 1000


def _sc_mesh():
    return plsc.VectorSubcoreMesh(core_axis_name="c", subcore_axis_name="s")


def _sc_prep(row, col, zeros, ones):
    @functools.partial(
        pl.kernel,
        mesh=_sc_mesh(),
        out_type=[
            jax.ShapeDtypeStruct((_E,), jnp.int32),
            jax.ShapeDtypeStruct((_NC, _NPAD, 128), jnp.float32),
        ],
        scratch_types=(
            [pltpu.VMEM((_K,), jnp.int32)] * (3 * _NBUF)
            + [pltpu.VMEM((_K, 128), jnp.float32),
               pltpu.VMEM_SHARED((_NPAD, 128), jnp.float32)]
            + [pltpu.SemaphoreType.DMA] * (4 * _NBUF)
        ),
    )
    def k(row_hbm, col_hbm, z_hbm, ones_hbm, colp_hbm, degp_hbm, *scr):
        idxb = scr[:3 * _NBUF]
        ob = scr[3 * _NBUF]
        table = scr[3 * _NBUF + 1]
        sems = scr[3 * _NBUF + 2:]
        bufs = tuple(
            (idxb[3 * b], idxb[3 * b + 1], idxb[3 * b + 2],
             sems[4 * b], sems[4 * b + 1], sems[4 * b + 2], sems[4 * b + 3])
            for b in range(_NBUF))
        c = lax.axis_index("c")
        s = lax.axis_index("s")
        base = (c * _NS + s) * _EPW

        def start_idx(i, b):
            rb, cb, _, sr, sc, _, _ = bufs[b]
            off = base + i * _K
            pltpu.async_copy(row_hbm.at[pl.ds(off, _K)], rb, sr)
            pltpu.async_copy(col_hbm.at[pl.ds(off, _K)], cb, sc)

        def do_chunk(i, b):
            rb, cb, cpb, sr, sc, so, ss = bufs[b]
            off = base + i * _K
            pltpu.make_async_copy(row_hbm.at[pl.ds(0, _K)], rb, sr).wait()
            pltpu.make_async_copy(col_hbm.at[pl.ds(0, _K)], cb, sc).wait()
            for j in range(_K // 16):
                sl = pl.ds(j * 16, 16)
                r = rb[sl]
                cc = cb[sl]
                cpb[sl] = jnp.where(r == cc, _DUMMY, cc)
            pltpu.async_copy(cpb, colp_hbm.at[pl.ds(off, _K)], so)
            pltpu.async_copy(ob, table.at[cpb], ss, add=True)

        def drain(b):
            _, _, cpb, _, _, so, ss = bufs[b]
            pltpu.make_async_copy(cpb, colp_hbm.at[pl.ds(0, _K)], so).wait()
            pltpu.make_async_copy(ones_hbm, ob, ss).wait()

        for b in range(_NBUF):
            start_idx(b, b)
        pltpu.sync_copy(z_hbm.at[pl.ds(s * _RPT, _RPT)],
                        table.at[pl.ds(s * _RPT, _RPT)])
        pltpu.sync_copy(ones_hbm, ob)
        plsc.subcore_barrier()
        nfull = (_NCHUNK - 1) // _NBUF

        def body(p, carry):
            for b in range(_NBUF):
                do_chunk(p * _NBUF + b, b)
            for b in range(_NBUF):
                drain(b)
                nxt = p * _NBUF + b + _NBUF
                if b == 0:
                    start_idx(nxt, b)
                else:
                    @pl.when(nxt < _NCHUNK)
                    def _():
                        start_idx(nxt, b)
            return carry

        lax.fori_loop(0, nfull, body, 0)
        for i in range(nfull * _NBUF, _NCHUNK):
            do_chunk(i, i % _NBUF)
        for i in range(nfull * _NBUF, _NCHUNK):
            drain(i % _NBUF)
        plsc.subcore_barrier()
        pltpu.sync_copy(table.at[pl.ds(s * _RPT, _RPT)],
                        degp_hbm.at[c, pl.ds(s * _RPT, _RPT)])

    return k(row, col, zeros, ones)


def _sc_scatter(y, row, colp, zeros, d, untiled=False):
    @functools.partial(
        pl.kernel,
        mesh=_sc_mesh(),
        compiler_params=(pltpu.CompilerParams(use_tc_tiling_on_sc=False)
                         if untiled else None),
        out_type=jax.ShapeDtypeStruct((_NC, _NPAD, d), jnp.float32),
        scratch_types=(
            [pltpu.VMEM((_K,), jnp.int32)] * (2 * _NBUF)
            + [pltpu.VMEM((_K, d), jnp.float32)] * _NBUF
            + [pltpu.VMEM_SHARED((_NPAD, d), jnp.float32)]
            + [pltpu.SemaphoreType.DMA] * (4 * _NBUF)
        ),
    )
    def k(y_hbm, row_hbm, colp_hbm, z_hbm, out_hbm, *scr):
        idxb = scr[:2 * _NBUF]
        gbs = scr[2 * _NBUF:3 * _NBUF]
        table = scr[3 * _NBUF]
        sems = scr[3 * _NBUF + 1:]
        bufs = tuple(
            (idxb[2 * b], idxb[2 * b + 1], gbs[b],
             sems[4 * b], sems[4 * b + 1], sems[4 * b + 2], sems[4 * b + 3])
            for b in range(_NBUF))
        c = lax.axis_index("c")
        s = lax.axis_index("s")
        base = (c * _NS + s) * _EPW

        def start_idx(i, b):
            rb, cpb, _, sr, sc, _, _ = bufs[b]
            off = base + i * _K
            pltpu.async_copy(row_hbm.at[pl.ds(off, _K)], rb, sr)
            pltpu.async_copy(colp_hbm.at[pl.ds(off, _K)], cpb, sc)

        def start_gather(b):
            rb, _, gb, sr, _, sg, _ = bufs[b]
            pltpu.make_async_copy(row_hbm.at[pl.ds(0, _K)], rb, sr).wait()
            pltpu.async_copy(y_hbm.at[rb], gb, sg)

        def start_scatter(b):
            _, cpb, gb, _, sc, sg, ss = bufs[b]
            pltpu.make_async_copy(y_hbm.at[pl.ds(0, _K)], gb, sg).wait()
            pltpu.make_async_copy(colp_hbm.at[pl.ds(0, _K)], cpb, sc).wait()
            pltpu.async_copy(gb, table.at[cpb], ss, add=True)

        def drain_scatter(b):
            _, _, gb, _, _, _, ss = bufs[b]
            pltpu.make_async_copy(y_hbm.at[pl.ds(0, _K)], gb, ss).wait()

        for b in range(_NBUF):
            start_idx(b, b)
        for b in range(_NBUF):
            start_gather(b)
        pltpu.sync_copy(z_hbm.at[pl.ds(s * _RPT, _RPT)],
                        table.at[pl.ds(s * _RPT, _RPT)])
        plsc.subcore_barrier()
        nfull = (_NCHUNK - 1) // _NBUF

        def body(p, carry):
            for b in range(_NBUF):
                start_scatter(b)
            for b in range(_NBUF):
                drain_scatter(b)
                nxt = p * _NBUF + b + _NBUF
                if b == 0:
                    start_idx(nxt, b)
                    start_gather(b)
                else:
                    @pl.when(nxt < _NCHUNK)
                    def _():
                        start_idx(nxt, b)
                        start_gather(b)
            return carry

        lax.fori_loop(0, nfull, body, 0)
        for i in range(nfull * _NBUF, _NCHUNK):
            start_scatter(i % _NBUF)
        for i in range(nfull * _NBUF, _NCHUNK):
            drain_scatter(i % _NBUF)
        plsc.subcore_barrier()
        pltpu.sync_copy(table.at[pl.ds(s * _RPT, _RPT)],
                        out_hbm.at[c, pl.ds(s * _RPT, _RPT)])

    return k(y, row, colp, zeros)


def _tc_stage1(degp, x, w):
    def body(degp_ref, x_ref, w_ref, y_ref, dis_ref):
        d0 = degp_ref[0]
        d1 = degp_ref[1]
        deg = d0[:, 0:1] + d1[:, 0:1] + 1.0
        dis = lax.rsqrt(deg)
        y_ref[...] = dis * jnp.dot(x_ref[...], w_ref[...],
                                   preferred_element_type=jnp.float32)
        dis_ref[...] = dis

    return pl.pallas_call(
        body,
        grid=(_N // _BM,),
        in_specs=[
            pl.BlockSpec((_NC, _BM, 128), lambda m: (0, m, 0)),
            pl.BlockSpec((_BM, 128), lambda m: (m, 0)),
            pl.BlockSpec((128, 128), lambda m: (0, 0)),
        ],
        out_specs=[
            pl.BlockSpec((_BM, 128), lambda m: (m, 0)),
            pl.BlockSpec((_BM, 1), lambda m: (m, 0)),
        ],
        out_shape=[
            jax.ShapeDtypeStruct((_N, 128), jnp.float32),
            jax.ShapeDtypeStruct((_N, 1), jnp.float32),
        ],
    )(degp, x, w)


def _tc_pweight(wp, a0, a1, ab, b0, b1, bb):
    def body(wp_r, a0_r, a1_r, ab_r, b0_r, b1_r, bb_r, out_ref):
        h1 = jnp.dot(wp_r[...], a0_r[...], preferred_element_type=jnp.float32)
        h1 = jnp.where(h1 >= 0, h1, 0.2 * h1)
        w1 = jnp.dot(h1, a1_r[...], preferred_element_type=jnp.float32) + ab_r[...]
        w1 = jnp.where(w1 >= 0, w1, 0.01 * w1)
        h2 = jnp.dot(wp_r[...], b0_r[...], preferred_element_type=jnp.float32)
        h2 = jnp.where(h2 >= 0, h2, 0.2 * h2)
        w2 = jnp.dot(h2, b1_r[...], preferred_element_type=jnp.float32) + bb_r[...]
        w2 = jnp.where(w2 >= 0, w2, 0.01 * w2)
        nsq = (jnp.sum(w1 * w1, axis=1, keepdims=True)
               + jnp.sum(w2 * w2, axis=1, keepdims=True))
        out_ref[...] = jnp.sqrt(nsq)

    return pl.pallas_call(
        body,
        out_shape=jax.ShapeDtypeStruct((4, 1), jnp.float32),
    )(wp, a0, a1, ab, b0, b1, bb)


def _tc_stage2(sp, y1, dis, b1, w2):
    def body(sp_ref, y1_ref, dis_ref, b1_ref, w2_ref, y2_ref):
        o = dis_ref[...] * (sp_ref[0] + sp_ref[1] + y1_ref[...]) + b1_ref[...]
        h = jnp.where(o > 0, o, jnp.exp(jnp.minimum(o, 0.0)) - 1.0)
        y2_ref[...] = dis_ref[...] * jnp.dot(h, w2_ref[...],
                                             preferred_element_type=jnp.float32)

    return pl.pallas_call(
        body,
        grid=(_N // _BM,),
        in_specs=[
            pl.BlockSpec((_NC, _BM, 128), lambda m: (0, m, 0)),
            pl.BlockSpec((_BM, 128), lambda m: (m, 0)),
            pl.BlockSpec((_BM, 1), lambda m: (m, 0)),
            pl.BlockSpec((1, 128), lambda m: (0, 0)),
            pl.BlockSpec((128, 64), lambda m: (0, 0)),
        ],
        out_specs=pl.BlockSpec((_BM, 64), lambda m: (m, 0)),
        out_shape=jax.ShapeDtypeStruct((_N, 64), jnp.float32),
    )(sp, y1, dis, b1, w2)


def _tc_stage3(sp, y2, dis, b2):
    def body(sp_ref, y2_ref, dis_ref, b2_ref, out_ref):
        t = sp_ref[0] + sp_ref[1] + y2_ref[...]
        o = dis_ref[...] * t + b2_ref[...]
        m = jnp.max(o, axis=1, keepdims=True)
        e = jnp.exp(o - m)
        out_ref[...] = (o - m) - jnp.log(jnp.sum(e, axis=1, keepdims=True))

    return pl.pallas_call(
        body,
        grid=(_N // _BM,),
        in_specs=[
            pl.BlockSpec((_NC, _BM, 64), lambda m: (0, m, 0)),
            pl.BlockSpec((_BM, 64), lambda m: (m, 0)),
            pl.BlockSpec((_BM, 1), lambda m: (m, 0)),
            pl.BlockSpec((1, 64), lambda m: (0, 0)),
        ],
        out_specs=pl.BlockSpec((_BM, 64), lambda m: (m, 0)),
        out_shape=jax.ShapeDtypeStruct((_N, 64), jnp.float32),
    )(sp, y2, dis, b2)


def kernel(x, edge_index, w_mul_p, lin1_w, bias1, mlp1_w0, mlp1_w1, mlp1_b1,
           lin2_w, bias2, mlp2_w0, mlp2_w1, mlp2_b1):
    zeros128 = jnp.zeros((_NPAD, 128), jnp.float32)
    zeros64 = jnp.zeros((_NPAD, 64), jnp.float32)
    ones_k = jnp.ones((_K, 128), jnp.float32)

    row = edge_index[0]
    col = edge_index[1]
    colp, degp = _sc_prep(row, col, zeros128, ones_k)
    y1, dis = _tc_stage1(degp, x, lin1_w)
    s1 = _sc_scatter(y1, row, colp, zeros128, 128)
    y2 = _tc_stage2(s1, y1, dis, bias1.reshape(1, 128), lin2_w)
    s2 = _sc_scatter(y2, row, colp, zeros64, 64, untiled=True)
    logp = _tc_stage3(s2, y2, dis, bias2.reshape(1, 64))
    norms = _tc_pweight(w_mul_p, mlp1_w0, mlp1_w1, mlp1_b1.reshape(1, 128),
                        mlp2_w0, mlp2_w1, mlp2_b1.reshape(1, 64)).reshape(4)
    return logp, norms
